# baseline (speedup 1.0000x reference)
import functools

import jax
import jax.numpy as jnp
from jax import lax
from jax.experimental import pallas as pl
from jax.experimental.pallas import tpu as pltpu
from jax.experimental.pallas import tpu_sc as plsc

NUM_A = 16384
NUM_B = 26
NUM_ROWS = NUM_A * NUM_B
DIM = 32
NC, NS = 2, 16
NW = NC * NS
ROWS_PER_W = NUM_ROWS // NW
CHUNK = 128
NCHUNK = ROWS_PER_W // CHUNK
NBUF = 2
MAIN = NCHUNK - NBUF


def _body(idx_hbm, table_hbm, out5_hbm, idx_v, rows_v, trans_v, g0, g1, s0, s1):
    gsem = (g0, g1)
    ssem = (s0, s1)
    w = lax.axis_index("s") * NC + lax.axis_index("c")
    pltpu.sync_copy(idx_hbm.at[w], idx_v)
    cbase = w * NCHUNK
    lane = lax.iota(jnp.int32, 16)

    def fire(jc, p):
        pltpu.async_copy(table_hbm.at[idx_v.at[jc]], rows_v.at[p], gsem[p])

    def drain_gather(jc, p):
        pltpu.make_async_copy(
            table_hbm.at[idx_v.at[jc]], rows_v.at[p], gsem[p]).wait()

    def store(jc, p):
        c = cbase + jc
        b1 = c // 128
        ac = c % 128
        for jr in range(4):
            pltpu.async_copy(
                trans_v.at[p].at[jr], out5_hbm.at[b1].at[jr].at[ac], ssem[p])

    def drain_store(jc, p):
        c = cbase + jc
        b1 = c // 128
        ac = c % 128
        for jr in range(4):
            pltpu.make_async_copy(
                trans_v.at[p].at[jr], out5_hbm.at[b1].at[jr].at[ac],
                ssem[p]).wait()

    def transpose(p):
        def jstep(j):
            jr = j // 8
            jj = j % 8
            col = jnp.full((16,), j, jnp.int32)
            for q in range(8):
                row = lane + (16 * q)
                v = plsc.load_gather(rows_v.at[p], [row, col])
                trans_v.at[p][jr, jj, pl.ds(16 * q, 16)] = v

        pl.loop(0, DIM)(jstep)

    def process(jc, p, fire_next, drain_prev):
        drain_gather(jc, p)
        if drain_prev:
            drain_store(jc - NBUF, p)
        transpose(p)
        if fire_next:
            fire(jc + NBUF, p)
        store(jc, p)

    for p in range(NBUF):
        fire(p, p)

    def grp(g):
        for p in range(NBUF):
            process(g + p, p, True, True)

    def grp_head(g):
        for p in range(NBUF):
            process(g + p, p, True, False)

    grp_head(0)
    pl.loop(NBUF, MAIN, step=NBUF)(grp)

    for p in range(NBUF):
        process(MAIN + p, p, False, True)
    for p in range(NBUF):
        drain_store(MAIN + p, p)


@jax.jit
def _run(idx3, table):
    k = pl.kernel(
        _body,
        out_type=jax.ShapeDtypeStruct((NUM_B, 4, 128, 8, 128), jnp.float32),
        mesh=plsc.VectorSubcoreMesh(core_axis_name="c", subcore_axis_name="s"),
        scratch_types=[
            pltpu.VMEM((NCHUNK, CHUNK), jnp.int32),
            pltpu.VMEM((NBUF, CHUNK, DIM), jnp.float32),
            pltpu.VMEM((NBUF, 4, 8, CHUNK), jnp.float32),
            pltpu.SemaphoreType.DMA,
            pltpu.SemaphoreType.DMA,
            pltpu.SemaphoreType.DMA,
            pltpu.SemaphoreType.DMA,
        ],
        compiler_params=pltpu.CompilerParams(use_tc_tiling_on_sc=False, needs_layout_passes=False),
    )
    return k(idx3, table)


def kernel(inputs, table):
    idx3 = inputs.astype(jnp.int32).T.reshape(NW, NCHUNK, CHUNK)
    out5 = _run(idx3, table)
    return out5.transpose(2, 4, 0, 1, 3).reshape(NUM_A, NUM_B, DIM)


# speedup vs baseline: 1.4126x; 1.4126x over previous
import functools

import jax
import jax.numpy as jnp
from jax import lax
from jax.experimental import pallas as pl
from jax.experimental.pallas import tpu as pltpu
from jax.experimental.pallas import tpu_sc as plsc

NUM_A = 16384
NUM_B = 26
NUM_ROWS = NUM_A * NUM_B
DIM = 32
NC, NS = 2, 16
NW = NC * NS
ROWS_PER_W = NUM_ROWS // NW
CHUNK = 128
NCHUNK = ROWS_PER_W // CHUNK
NBUF = 2
MAIN = NCHUNK - NBUF


def _body(idx_hbm, table_hbm, out5_hbm, idx_v, rows_v, trans_v, g0, g1, s0, s1):
    gsem = (g0, g1)
    ssem = (s0, s1)
    w = lax.axis_index("s") * NC + lax.axis_index("c")
    pltpu.sync_copy(idx_hbm.at[w], idx_v)
    cbase = w * NCHUNK
    lane = lax.iota(jnp.int32, 16)

    def fire(jc, p):
        pltpu.async_copy(table_hbm.at[idx_v.at[jc]], rows_v.at[p], gsem[p])

    def drain_gather(jc, p):
        pltpu.make_async_copy(
            table_hbm.at[idx_v.at[jc]], rows_v.at[p], gsem[p]).wait()

    def store(jc, p):
        c = cbase + jc
        b1 = c // 128
        ac = c % 128
        for jr in range(4):
            pltpu.async_copy(
                trans_v.at[p].at[pl.ds(8 * jr, 8), pl.ds(0, CHUNK)],
                out5_hbm.at[b1].at[jr].at[ac], ssem[p])

    def drain_store(jc, p):
        c = cbase + jc
        b1 = c // 128
        ac = c % 128
        for jr in range(4):
            pltpu.make_async_copy(
                trans_v.at[p].at[pl.ds(8 * jr, 8), pl.ds(0, CHUNK)],
                out5_hbm.at[b1].at[jr].at[ac], ssem[p]).wait()

    lane16 = lane + 16

    def transpose(p):
        def astep(a0):
            for u in range(8):
                aa = a0 + u
                col = jnp.full((16,), aa, jnp.int32)
                v0 = rows_v.at[p][aa, pl.ds(0, 16)]
                v1 = rows_v.at[p][aa, pl.ds(16, 16)]
                plsc.store_scatter(trans_v.at[p], [lane, col], v0)
                plsc.store_scatter(trans_v.at[p], [lane16, col], v1)

        pl.loop(0, CHUNK, step=8)(astep)

    def process(jc, p, fire_next, drain_prev):
        drain_gather(jc, p)
        if drain_prev:
            drain_store(jc - NBUF, p)
        transpose(p)
        if fire_next:
            fire(jc + NBUF, p)
        store(jc, p)

    for p in range(NBUF):
        fire(p, p)

    def grp(g):
        for p in range(NBUF):
            process(g + p, p, True, True)

    def grp_head(g):
        for p in range(NBUF):
            process(g + p, p, True, False)

    grp_head(0)
    pl.loop(NBUF, MAIN, step=NBUF)(grp)

    for p in range(NBUF):
        process(MAIN + p, p, False, True)
    for p in range(NBUF):
        drain_store(MAIN + p, p)


@jax.jit
def _run(idx3, table):
    k = pl.kernel(
        _body,
        out_type=jax.ShapeDtypeStruct((NUM_B, 4, 128, 8, 128), jnp.float32),
        mesh=plsc.VectorSubcoreMesh(core_axis_name="c", subcore_axis_name="s"),
        scratch_types=[
            pltpu.VMEM((NCHUNK, CHUNK), jnp.int32),
            pltpu.VMEM((NBUF, CHUNK, DIM), jnp.float32),
            pltpu.VMEM((NBUF, DIM, 129), jnp.float32),
            pltpu.SemaphoreType.DMA,
            pltpu.SemaphoreType.DMA,
            pltpu.SemaphoreType.DMA,
            pltpu.SemaphoreType.DMA,
        ],
        compiler_params=pltpu.CompilerParams(use_tc_tiling_on_sc=False, needs_layout_passes=False),
    )
    return k(idx3, table)


def kernel(inputs, table):
    idx3 = inputs.astype(jnp.int32).T.reshape(NW, NCHUNK, CHUNK)
    out5 = _run(idx3, table)
    return out5.transpose(2, 4, 0, 1, 3).reshape(NUM_A, NUM_B, DIM)
